# baseline (device time: 110501 ns/iter reference)
import jax
import jax.numpy as jnp
from jax import lax
from jax.experimental import pallas as pl
from jax.experimental.pallas import tpu as pltpu

M = 2048
D = 2048
H = M // 2
C = 16
CR = H // C


def kernel(partial, resid, gamma):
    def body(partial_ref, resid_ref, gamma_ref, out_ref,
             yhalf, xhalf, sbuf, pbuf, rbuf, obuf,
             y_send, y_recv, x_send, x_recv,
             stage_sem, pin_sem, rin_sem, out_sem):
        my_x = lax.axis_index("x")
        my_y = lax.axis_index("y")
        my_z = lax.axis_index("z")
        y_peer = (my_x, 1 - my_y, my_z)
        x_peer = (1 - my_x, my_y, my_z)
        half0 = my_x * H
        other0 = (1 - my_x) * H

        barrier_sem = pltpu.get_barrier_semaphore()
        for nbr in (y_peer, x_peer):
            pl.semaphore_signal(
                barrier_sem, inc=1,
                device_id=nbr, device_id_type=pl.DeviceIdType.MESH,
            )
        pl.semaphore_wait(barrier_sem, 2)

        stage_cp = [
            pltpu.make_async_copy(
                partial_ref.at[0, pl.ds(half0 + c * CR, CR)],
                sbuf.at[pl.ds(c * CR, CR)],
                stage_sem.at[c],
            )
            for c in range(C)
        ]
        y_rdma = [
            pltpu.make_async_remote_copy(
                src_ref=sbuf.at[pl.ds(c * CR, CR)],
                dst_ref=yhalf.at[pl.ds(c * CR, CR)],
                send_sem=y_send.at[c],
                recv_sem=y_recv.at[c],
                device_id=y_peer,
                device_id_type=pl.DeviceIdType.MESH,
            )
            for c in range(C)
        ]
        x_rdma = [
            pltpu.make_async_remote_copy(
                src_ref=yhalf.at[pl.ds(c * CR, CR)],
                dst_ref=xhalf.at[pl.ds(c * CR, CR)],
                send_sem=x_send.at[c],
                recv_sem=x_recv.at[c],
                device_id=x_peer,
                device_id_type=pl.DeviceIdType.MESH,
            )
            for c in range(C)
        ]

        for c in range(C):
            stage_cp[c].start()
        for c in range(C):
            stage_cp[c].wait()
            y_rdma[c].start()

        order = [("y", 0)]
        for c in range(1, C):
            order.append(("y", c))
            order.append(("x", c - 1))
        order.append(("x", C - 1))

        def rows0(kind, c):
            return (half0 if kind == "y" else other0) + c * CR

        def stage_in(j):
            kind, c = order[j]
            s = j % 2
            pin = pltpu.make_async_copy(
                partial_ref.at[0, pl.ds(rows0(kind, c), CR)], pbuf.at[s],
                pin_sem.at[s],
            )
            rin = pltpu.make_async_copy(
                resid_ref.at[pl.ds(rows0(kind, c), CR)], rbuf.at[s],
                rin_sem.at[s],
            )
            pin.start()
            rin.start()
            return pin, rin

        out_copies = [None] * (2 * C)
        stage = stage_in(0)
        for j in range(2 * C):
            kind, c = order[j]
            if kind == "y":
                y_rdma[c].wait_recv()
                x_rdma[c].start()
                comm = yhalf.at[pl.ds(c * CR, CR)]
            else:
                x_rdma[c].wait_recv()
                comm = xhalf.at[pl.ds(c * CR, CR)]

            pin, rin = stage
            if j + 1 < 2 * C:
                stage = stage_in(j + 1)
            pin.wait()
            rin.wait()
            s = j % 2
            if j >= 2:
                out_copies[j - 2].wait()

            y = pbuf[s] + comm[...] + rbuf[s]
            inv = lax.rsqrt(jnp.mean(y * y, axis=-1, keepdims=True) + 1e-6)
            obuf[s] = y * inv * gamma_ref[...]

            oc = pltpu.make_async_copy(
                obuf.at[s], out_ref.at[pl.ds(rows0(kind, c), CR)],
                out_sem.at[s],
            )
            oc.start()
            out_copies[j] = oc

        out_copies[2 * C - 2].wait()
        out_copies[2 * C - 1].wait()
        for c in range(C):
            y_rdma[c].wait_send()
            x_rdma[c].wait_send()

    return pl.pallas_call(
        body,
        out_shape=jax.ShapeDtypeStruct((M, D), jnp.float32),
        in_specs=[
            pl.BlockSpec(memory_space=pltpu.MemorySpace.HBM),
            pl.BlockSpec(memory_space=pltpu.MemorySpace.HBM),
            pl.BlockSpec(memory_space=pltpu.VMEM),
        ],
        out_specs=pl.BlockSpec(memory_space=pltpu.MemorySpace.HBM),
        scratch_shapes=[
            pltpu.VMEM((H, D), jnp.float32),
            pltpu.VMEM((H, D), jnp.float32),
            pltpu.VMEM((H, D), jnp.float32),
            pltpu.VMEM((2, CR, D), jnp.float32),
            pltpu.VMEM((2, CR, D), jnp.float32),
            pltpu.VMEM((2, CR, D), jnp.float32),
            pltpu.SemaphoreType.DMA((C,)),
            pltpu.SemaphoreType.DMA((C,)),
            pltpu.SemaphoreType.DMA((C,)),
            pltpu.SemaphoreType.DMA((C,)),
            pltpu.SemaphoreType.DMA((C,)),
            pltpu.SemaphoreType.DMA((2,)),
            pltpu.SemaphoreType.DMA((2,)),
            pltpu.SemaphoreType.DMA((2,)),
        ],
        compiler_params=pltpu.CompilerParams(collective_id=0),
    )(partial, resid, gamma.reshape(1, D))


# device time: 92233 ns/iter; 1.1981x vs baseline; 1.1981x over previous
import jax
import jax.numpy as jnp
from jax import lax
from jax.experimental import pallas as pl
from jax.experimental.pallas import tpu as pltpu

M = 2048
D = 2048
Q = M // 4
C = 8
CR = Q // C


def kernel(partial, resid, gamma):
    def body(partial_ref, resid_ref, gamma_ref, out_ref,
             peer_sh, sbuf, pbuf, rbuf, obuf,
             y_send, y_recv, xf_send, xf_recv, zf_send, zf_recv,
             xd_send, xd_recv, zd_send, zd_recv,
             stage_sem, pin_sem, rin_sem, out_sem):
        my_x = lax.axis_index("x")
        my_y = lax.axis_index("y")
        my_z = lax.axis_index("z")
        y_peer = (my_x, 1 - my_y, my_z)
        x_peer = (1 - my_x, my_y, my_z)
        z_peer = (my_x, my_y, 1 - my_z)

        qm0 = (2 * my_x + my_z) * Q
        qx0 = (2 * (1 - my_x) + my_z) * Q
        qz0 = (2 * my_x + (1 - my_z)) * Q
        qd0 = (2 * (1 - my_x) + (1 - my_z)) * Q

        barrier_sem = pltpu.get_barrier_semaphore()
        for nbr in (y_peer, x_peer, z_peer):
            pl.semaphore_signal(
                barrier_sem, inc=1,
                device_id=nbr, device_id_type=pl.DeviceIdType.MESH,
            )
        pl.semaphore_wait(barrier_sem, 3)

        def chunk(base, c):
            return pl.ds(base + c * CR, CR)

        def remote(src, dst, send_sem, recv_sem, dev):
            return pltpu.make_async_remote_copy(
                src_ref=src, dst_ref=dst, send_sem=send_sem,
                recv_sem=recv_sem, device_id=dev,
                device_id_type=pl.DeviceIdType.MESH,
            )

        stage_cp = [
            pltpu.make_async_copy(
                partial_ref.at[0, chunk(qm0, c)], sbuf.at[pl.ds(c * CR, CR)],
                stage_sem.at[c],
            )
            for c in range(C)
        ]
        y_ex = [
            remote(sbuf.at[pl.ds(c * CR, CR)], peer_sh.at[chunk(qm0, c)],
                   y_send.at[c], y_recv.at[c], y_peer)
            for c in range(C)
        ]
        xf = [
            remote(peer_sh.at[chunk(qm0, c)], peer_sh.at[chunk(qm0, c)],
                   xf_send.at[c], xf_recv.at[c], x_peer)
            for c in range(C)
        ]
        zf = [
            remote(peer_sh.at[chunk(qm0, c)], peer_sh.at[chunk(qm0, c)],
                   zf_send.at[c], zf_recv.at[c], z_peer)
            for c in range(C)
        ]
        xd = [
            remote(peer_sh.at[chunk(qz0, c)], peer_sh.at[chunk(qz0, c)],
                   xd_send.at[c], xd_recv.at[c], x_peer)
            for c in range(C // 2)
        ]
        zd = [
            remote(peer_sh.at[chunk(qx0, 4 + c)], peer_sh.at[chunk(qx0, 4 + c)],
                   zd_send.at[c], zd_recv.at[c], z_peer)
            for c in range(C // 2)
        ]

        for c in range(C):
            stage_cp[c].start()
        for c in range(C):
            stage_cp[c].wait()
            y_ex[c].start()

        sched = []
        for c in range(C):
            sched.append((y_ex[c], [xf[c], zf[c]], qm0, c))
        for c in range(C):
            sched.append((xf[c], [zd[c - 4]] if c >= 4 else [], qx0, c))
            sched.append((zf[c], [xd[c]] if c < 4 else [], qz0, c))
        for c in range(C // 2):
            sched.append((xd[c], [], qd0, c))
        for c in range(C // 2):
            sched.append((zd[c], [], qd0, 4 + c))
        n = len(sched)

        def stage_in(j):
            _, _, base, c = sched[j]
            s = j % 2
            pin = pltpu.make_async_copy(
                partial_ref.at[0, chunk(base, c)], pbuf.at[s], pin_sem.at[s],
            )
            rin = pltpu.make_async_copy(
                resid_ref.at[chunk(base, c)], rbuf.at[s], rin_sem.at[s],
            )
            pin.start()
            rin.start()
            return pin, rin

        out_copies = [None] * n
        stage = stage_in(0)
        for j in range(n):
            rdma, starts, base, c = sched[j]
            rdma.wait_recv()
            for r in starts:
                r.start()

            pin, rin = stage
            if j + 1 < n:
                stage = stage_in(j + 1)
            pin.wait()
            rin.wait()
            s = j % 2
            if j >= 2:
                out_copies[j - 2].wait()

            y = pbuf[s] + peer_sh[chunk(base, c)] + rbuf[s]
            inv = lax.rsqrt(jnp.mean(y * y, axis=-1, keepdims=True) + 1e-6)
            obuf[s] = y * inv * gamma_ref[...]

            oc = pltpu.make_async_copy(
                obuf.at[s], out_ref.at[chunk(base, c)], out_sem.at[s],
            )
            oc.start()
            out_copies[j] = oc

        out_copies[n - 2].wait()
        out_copies[n - 1].wait()
        for c in range(C):
            y_ex[c].wait_send()
            xf[c].wait_send()
            zf[c].wait_send()
        for c in range(C // 2):
            xd[c].wait_send()
            zd[c].wait_send()

    return pl.pallas_call(
        body,
        out_shape=jax.ShapeDtypeStruct((M, D), jnp.float32),
        in_specs=[
            pl.BlockSpec(memory_space=pltpu.MemorySpace.HBM),
            pl.BlockSpec(memory_space=pltpu.MemorySpace.HBM),
            pl.BlockSpec(memory_space=pltpu.VMEM),
        ],
        out_specs=pl.BlockSpec(memory_space=pltpu.MemorySpace.HBM),
        scratch_shapes=[
            pltpu.VMEM((M, D), jnp.float32),
            pltpu.VMEM((Q, D), jnp.float32),
            pltpu.VMEM((2, CR, D), jnp.float32),
            pltpu.VMEM((2, CR, D), jnp.float32),
            pltpu.VMEM((2, CR, D), jnp.float32),
            pltpu.SemaphoreType.DMA((C,)),
            pltpu.SemaphoreType.DMA((C,)),
            pltpu.SemaphoreType.DMA((C,)),
            pltpu.SemaphoreType.DMA((C,)),
            pltpu.SemaphoreType.DMA((C,)),
            pltpu.SemaphoreType.DMA((C,)),
            pltpu.SemaphoreType.DMA((C // 2,)),
            pltpu.SemaphoreType.DMA((C // 2,)),
            pltpu.SemaphoreType.DMA((C // 2,)),
            pltpu.SemaphoreType.DMA((C // 2,)),
            pltpu.SemaphoreType.DMA((C,)),
            pltpu.SemaphoreType.DMA((2,)),
            pltpu.SemaphoreType.DMA((2,)),
            pltpu.SemaphoreType.DMA((2,)),
        ],
        compiler_params=pltpu.CompilerParams(collective_id=0),
    )(partial, resid, gamma.reshape(1, D))
